# baseline (speedup 1.0000x reference)
import functools

import jax
import jax.numpy as jnp
from jax import lax
from jax.experimental import pallas as pl
from jax.experimental.pallas import tpu as pltpu
from jax.experimental.pallas import tpu_sc as plsc

N = 50000
E = 800000
H = 64

NC = 2
NS = 16
CHUNK = 128
HALF = N // NC
HP = HALF + 88
RPT = HP // NS

CPT = 396
NCHUNK = CPT * NS
EPAD = NCHUNK * CHUNK

NBA = 3
QIA = CPT // NBA

BFB = 12
NBB = 3
QIB = CPT // (BFB * NBB)

CW = 8


def _enc_body(x_ref, w1_ref, b1_ref, w2_ref, b2_ref, h_ref):
    a = jnp.maximum(jnp.dot(x_ref[...], w1_ref[...],
                            preferred_element_type=jnp.float32) + b1_ref[...], 0.0)
    h_ref[...] = jnp.dot(a, w2_ref[...],
                         preferred_element_type=jnp.float32) + b2_ref[...]


def _mlp_body(h_ref, sums_ref, cnt_ref, wf1_ref, bf1_ref, wf2_ref, bf2_ref,
              wh1_ref, bh1_ref, wh2_ref, bh2_ref, out_ref):
    cnt = cnt_ref[...][:, 0:1]
    h = h_ref[...] + sums_ref[...] / jnp.maximum(cnt, 1.0)
    h = jnp.maximum(jnp.dot(h, wf1_ref[...],
                            preferred_element_type=jnp.float32) + bf1_ref[...], 0.0)
    h = jnp.maximum(jnp.dot(h, wf2_ref[...],
                            preferred_element_type=jnp.float32) + bf2_ref[...], 0.0)
    hid = jnp.maximum(jnp.dot(h, wh1_ref[...],
                              preferred_element_type=jnp.float32) + bh1_ref[...], 0.0)
    out_ref[...] = jnp.dot(hid, wh2_ref[...],
                           preferred_element_type=jnp.float32) + bh2_ref[...]


def _remap_src(srcb, idxb, p, j, base):
    for i in range(CHUNK // 16):
        sl = pl.ds(i * 16, 16)
        rel = srcb[p, j, sl] - base
        ok = (rel >= 0) & (rel < HALF)
        idxb[p, j, sl] = jnp.where(ok, rel, HALF)


def _sc_sums_body(h_hbm, src_hbm, dst_hbm, z64_hbm,
                  sums_out,
                  srcb, dstb, idxb, msgb, sums_sh,
                  g0, g1, g2, s0, s1, s2):
    c = lax.axis_index("c")
    s = lax.axis_index("s")
    base = c * HALF
    gsems = [g0, g1, g2]
    ssems = [s0, s1, s2]

    pltpu.sync_copy(z64_hbm, sums_sh.at[pl.ds(s * RPT, RPT)])
    plsc.subcore_barrier()

    def load_idx(g, p):
        ci = s * CPT + g
        pltpu.sync_copy(src_hbm.at[pl.ds(ci, 1)], srcb.at[p])
        pltpu.sync_copy(dst_hbm.at[pl.ds(ci, 1)], dstb.at[p])
        _remap_src(srcb, idxb, p, 0, base)

    def fire_gather(p):
        pltpu.async_copy(h_hbm.at[dstb.at[p, 0]], msgb.at[p], gsems[p])

    def drain_gather(p):
        pltpu.make_async_copy(h_hbm.at[dstb.at[p, 0]], msgb.at[p],
                              gsems[p]).wait()

    def fire_scatter(p):
        pltpu.async_copy(msgb.at[p], sums_sh.at[idxb.at[p, 0]],
                         ssems[p], add=True)

    def drain_scatter(p):
        pltpu.make_async_copy(msgb.at[p], sums_sh.at[idxb.at[p, 0]],
                              ssems[p]).wait()

    load_idx(0, 0)
    fire_gather(0)
    load_idx(1, 1)
    fire_gather(1)

    def q_body(q, carry):
        for p in range(NBA):
            g = NBA * q + p
            drain_gather(p)
            fire_scatter(p)
            nxt = g + 2
            pn = (p + 2) % NBA

            @pl.when(nxt < CPT)
            def _prefetch(nxt=nxt, pn=pn):
                @pl.when(nxt >= NBA)
                def _reuse(pn=pn):
                    drain_scatter(pn)
                load_idx(nxt, pn)
                fire_gather(pn)
        return carry

    lax.fori_loop(0, QIA, q_body, 0)
    for p in range(NBA):
        drain_scatter(p)
    plsc.subcore_barrier()

    ob = c * HP + s * RPT
    pltpu.sync_copy(sums_sh.at[pl.ds(s * RPT, RPT)], sums_out.at[pl.ds(ob, RPT)])


def _sc_cnt_body(src_hbm, zcw_hbm, ones_hbm,
                 cnt_out,
                 srcb, idxb, onesb, cnt_sh,
                 s0, s1, s2):
    c = lax.axis_index("c")
    s = lax.axis_index("s")
    base = c * HALF
    ssems = [s0, s1, s2]

    pltpu.sync_copy(zcw_hbm, cnt_sh.at[pl.ds(s * RPT, RPT)])
    pltpu.sync_copy(ones_hbm, onesb)
    plsc.subcore_barrier()

    def load_idx(g, p):
        ci = s * CPT + g * BFB
        pltpu.sync_copy(src_hbm.at[pl.ds(ci, BFB)], srcb.at[p])
        for j in range(BFB):
            _remap_src(srcb, idxb, p, j, base)

    def fire_scatters(p):
        for j in range(BFB):
            pltpu.async_copy(onesb, cnt_sh.at[idxb.at[p, j]],
                             ssems[p], add=True)

    def drain_scatters(p):
        for j in range(BFB):
            pltpu.make_async_copy(onesb, cnt_sh.at[idxb.at[p, j]],
                                  ssems[p]).wait()

    nbatch = CPT // BFB
    load_idx(0, 0)
    load_idx(1, 1)

    def q_body(q, carry):
        for p in range(NBB):
            g = NBB * q + p
            fire_scatters(p)
            nxt = g + 2
            pn = (p + 2) % NBB

            @pl.when(nxt < nbatch)
            def _prefetch(nxt=nxt, pn=pn):
                @pl.when(nxt >= NBB)
                def _reuse(pn=pn):
                    drain_scatters(pn)
                load_idx(nxt, pn)
        return carry

    lax.fori_loop(0, QIB, q_body, 0)
    for p in range(NBB):
        drain_scatters(p)
    plsc.subcore_barrier()

    ob = c * HP + s * RPT
    pltpu.sync_copy(cnt_sh.at[pl.ds(s * RPT, RPT)], cnt_out.at[pl.ds(ob, RPT)])


_sc_sums = functools.partial(
    pl.kernel,
    out_type=jax.ShapeDtypeStruct((NC * HP, H), jnp.float32),
    mesh=plsc.VectorSubcoreMesh(core_axis_name="c", subcore_axis_name="s"),
    compiler_params=pltpu.CompilerParams(use_tc_tiling_on_sc=False),
    scratch_types=[
        pltpu.VMEM((NBA, 1, CHUNK), jnp.int32),
        pltpu.VMEM((NBA, 1, CHUNK), jnp.int32),
        pltpu.VMEM((NBA, 1, CHUNK), jnp.int32),
        pltpu.VMEM((NBA, CHUNK, H), jnp.float32),
        pltpu.VMEM_SHARED((HP, H), jnp.float32),
    ] + [pltpu.SemaphoreType.DMA] * 6,
)(_sc_sums_body)


_sc_cnt = functools.partial(
    pl.kernel,
    out_type=jax.ShapeDtypeStruct((NC * HP, CW), jnp.float32),
    mesh=plsc.VectorSubcoreMesh(core_axis_name="c", subcore_axis_name="s"),
    compiler_params=pltpu.CompilerParams(use_tc_tiling_on_sc=False),
    scratch_types=[
        pltpu.VMEM((NBB, BFB, CHUNK), jnp.int32),
        pltpu.VMEM((NBB, BFB, CHUNK), jnp.int32),
        pltpu.VMEM((CHUNK, CW), jnp.float32),
        pltpu.VMEM_SHARED((HP, CW), jnp.float32),
    ] + [pltpu.SemaphoreType.DMA] * 3,
)(_sc_cnt_body)


BLK = 2000


def _row_spec(w):
    return pl.BlockSpec((BLK, w), lambda i: (i, 0))


def _full_spec(shape):
    return pl.BlockSpec(shape, lambda i: (0,) * len(shape))


def kernel(x, adj, W_enc1, b_enc1, W_enc2, b_enc2, W_fc1, b_fc1, W_fc2, b_fc2,
           W_s1, b_s1, W_s2, b_s2, W_t1, b_t1, W_t2, b_t2):
    f = x.shape[1]

    h = pl.pallas_call(
        _enc_body,
        grid=(N // BLK,),
        in_specs=[
            _row_spec(f),
            _full_spec((f, H)), _full_spec((1, H)),
            _full_spec((H, H)), _full_spec((1, H)),
        ],
        out_specs=_row_spec(H),
        out_shape=jax.ShapeDtypeStruct((N, H), jnp.float32),
    )(x, W_enc1, b_enc1.reshape(1, H), W_enc2, b_enc2.reshape(1, H))

    pad = EPAD - E
    src = jnp.concatenate([adj[0], jnp.full((pad,), N, jnp.int32)]).reshape(NCHUNK, CHUNK)
    dst = jnp.concatenate([adj[1], jnp.zeros((pad,), jnp.int32)]).reshape(NCHUNK, CHUNK)
    z64 = jnp.zeros((RPT, H), jnp.float32)
    zcw = jnp.zeros((RPT, CW), jnp.float32)
    ones = jnp.ones((CHUNK, CW), jnp.float32)
    sums_p = _sc_sums(h, src, dst, z64)
    cnt_p = _sc_cnt(src, zcw, ones)
    sums = jnp.concatenate([sums_p[:HALF], sums_p[HP:HP + HALF]], axis=0)
    cnt = jnp.concatenate([cnt_p[:HALF], cnt_p[HP:HP + HALF]], axis=0)

    wh1 = jnp.concatenate([W_s1, W_t1], axis=1)
    bh1 = jnp.concatenate([b_s1, b_t1]).reshape(1, 64)
    wh2 = jnp.zeros((64, 8), jnp.float32)
    wh2 = wh2.at[:32, 0:1].set(W_s2).at[32:, 1:5].set(W_t2)
    bh2 = jnp.zeros((1, 8), jnp.float32)
    bh2 = bh2.at[0, 0:1].set(b_s2).at[0, 1:5].set(b_t2)

    out8 = pl.pallas_call(
        _mlp_body,
        grid=(N // BLK,),
        in_specs=[
            _row_spec(H), _row_spec(H), _row_spec(CW),
            _full_spec((H, H)), _full_spec((1, H)),
            _full_spec((H, H)), _full_spec((1, H)),
            _full_spec((H, 64)), _full_spec((1, 64)),
            _full_spec((64, 8)), _full_spec((1, 8)),
        ],
        out_specs=_row_spec(8),
        out_shape=jax.ShapeDtypeStruct((N, 8), jnp.float32),
    )(h, sums, cnt, W_fc1, b_fc1.reshape(1, H), W_fc2, b_fc2.reshape(1, H),
      wh1, bh1, wh2, bh2)

    return out8[:, 0], out8[:, 1:5]


# speedup vs baseline: 1.6844x; 1.6844x over previous
import functools

import jax
import jax.numpy as jnp
from jax import lax
from jax.experimental import pallas as pl
from jax.experimental.pallas import tpu as pltpu
from jax.experimental.pallas import tpu_sc as plsc

N = 50000
E = 800000
H = 64
HA = H + 8

NC = 2
NS = 16
CHUNK = 112
HALF = N // NC
HP = HALF + 88
RPT = HP // NS

NBA = 2
CPT = 448
QIA = CPT // NBA
NCHUNK = CPT * NS
EPAD = NCHUNK * CHUNK


def _enc_body(x_ref, w1_ref, b1_ref, w2_ref, b2_ref, h_ref):
    a = jnp.maximum(jnp.dot(x_ref[...], w1_ref[...],
                            preferred_element_type=jnp.float32) + b1_ref[...], 0.0)
    h = jnp.dot(a, w2_ref[...], preferred_element_type=jnp.float32) + b2_ref[...]
    h_ref[...] = jnp.concatenate(
        [h, jnp.ones((h.shape[0], HA - H), jnp.float32)], axis=1)


def _mlp_body(h_ref, acc_ref, wf1_ref, bf1_ref, wf2_ref, bf2_ref,
              wh1_ref, bh1_ref, wh2_ref, bh2_ref, out_ref):
    cnt = acc_ref[...][:, H:H + 1]
    h = h_ref[...][:, :H] + acc_ref[...][:, :H] / jnp.maximum(cnt, 1.0)
    h = jnp.maximum(jnp.dot(h, wf1_ref[...],
                            preferred_element_type=jnp.float32) + bf1_ref[...], 0.0)
    h = jnp.maximum(jnp.dot(h, wf2_ref[...],
                            preferred_element_type=jnp.float32) + bf2_ref[...], 0.0)
    hid = jnp.maximum(jnp.dot(h, wh1_ref[...],
                              preferred_element_type=jnp.float32) + bh1_ref[...], 0.0)
    out_ref[...] = jnp.dot(hid, wh2_ref[...],
                           preferred_element_type=jnp.float32) + bh2_ref[...]


def _sc_agg_body(h_hbm, src_hbm, dst_hbm, zrow_hbm,
                 acc_out,
                 srcb, dstb, idxb, msgb, acc_sh,
                 g0, g1, s0, s1):
    c = lax.axis_index("c")
    s = lax.axis_index("s")
    base = c * HALF
    gsems = [g0, g1]
    ssems = [s0, s1]

    pltpu.sync_copy(zrow_hbm, acc_sh.at[pl.ds(s * RPT, RPT)])
    plsc.subcore_barrier()

    def load_idx(g, p):
        ci = s * CPT + g
        pltpu.sync_copy(src_hbm.at[pl.ds(ci, 1)], srcb.at[p])
        pltpu.sync_copy(dst_hbm.at[pl.ds(ci, 1)], dstb.at[p])
        for i in range(CHUNK // 16):
            sl = pl.ds(i * 16, 16)
            rel = srcb[p, 0, sl] - base
            ok = (rel >= 0) & (rel < HALF)
            idxb[p, 0, sl] = jnp.where(ok, rel, HALF)

    def fire_gather(p):
        pltpu.async_copy(h_hbm.at[dstb.at[p, 0]], msgb.at[p], gsems[p])

    def drain_gather(p):
        pltpu.make_async_copy(h_hbm.at[dstb.at[p, 0]], msgb.at[p],
                              gsems[p]).wait()

    def fire_scatter(p):
        pltpu.async_copy(msgb.at[p], acc_sh.at[idxb.at[p, 0]],
                         ssems[p], add=True)

    def drain_scatter(p):
        pltpu.make_async_copy(msgb.at[p], acc_sh.at[idxb.at[p, 0]],
                              ssems[p]).wait()

    load_idx(0, 0)
    fire_gather(0)
    load_idx(1, 1)
    fire_gather(1)

    def q_body(q, carry):
        for p in range(NBA):
            g = NBA * q + p
            drain_gather(p)
            fire_scatter(p)

            @pl.when(g + 2 < CPT)
            def _prefetch(g=g, p=p):
                drain_scatter(p)
                load_idx(g + 2, p)
                fire_gather(p)
        return carry

    lax.fori_loop(0, QIA, q_body, 0)
    for p in range(NBA):
        drain_scatter(p)
    plsc.subcore_barrier()

    ob = c * HP + s * RPT
    pltpu.sync_copy(acc_sh.at[pl.ds(s * RPT, RPT)], acc_out.at[pl.ds(ob, RPT)])


_sc_agg = functools.partial(
    pl.kernel,
    out_type=jax.ShapeDtypeStruct((NC * HP, HA), jnp.float32),
    mesh=plsc.VectorSubcoreMesh(core_axis_name="c", subcore_axis_name="s"),
    compiler_params=pltpu.CompilerParams(use_tc_tiling_on_sc=False),
    scratch_types=[
        pltpu.VMEM((NBA, 1, CHUNK), jnp.int32),
        pltpu.VMEM((NBA, 1, CHUNK), jnp.int32),
        pltpu.VMEM((NBA, 1, CHUNK), jnp.int32),
        pltpu.VMEM((NBA, CHUNK, HA), jnp.float32),
        pltpu.VMEM_SHARED((HP, HA), jnp.float32),
    ] + [pltpu.SemaphoreType.DMA] * 4,
)(_sc_agg_body)


BLK = 2000


def _row_spec(w):
    return pl.BlockSpec((BLK, w), lambda i: (i, 0))


def _full_spec(shape):
    return pl.BlockSpec(shape, lambda i: (0,) * len(shape))


def kernel(x, adj, W_enc1, b_enc1, W_enc2, b_enc2, W_fc1, b_fc1, W_fc2, b_fc2,
           W_s1, b_s1, W_s2, b_s2, W_t1, b_t1, W_t2, b_t2):
    f = x.shape[1]

    haug = pl.pallas_call(
        _enc_body,
        grid=(N // BLK,),
        in_specs=[
            _row_spec(f),
            _full_spec((f, H)), _full_spec((1, H)),
            _full_spec((H, H)), _full_spec((1, H)),
        ],
        out_specs=_row_spec(HA),
        out_shape=jax.ShapeDtypeStruct((N, HA), jnp.float32),
    )(x, W_enc1, b_enc1.reshape(1, H), W_enc2, b_enc2.reshape(1, H))

    pad = EPAD - E
    src = jnp.concatenate([adj[0], jnp.full((pad,), N, jnp.int32)]).reshape(NCHUNK, CHUNK)
    dst = jnp.concatenate([adj[1], jnp.zeros((pad,), jnp.int32)]).reshape(NCHUNK, CHUNK)
    zrow = jnp.zeros((RPT, HA), jnp.float32)
    acc_p = _sc_agg(haug, src, dst, zrow)
    acc = jnp.concatenate([acc_p[:HALF], acc_p[HP:HP + HALF]], axis=0)

    wh1 = jnp.concatenate([W_s1, W_t1], axis=1)
    bh1 = jnp.concatenate([b_s1, b_t1]).reshape(1, 64)
    wh2 = jnp.zeros((64, 8), jnp.float32)
    wh2 = wh2.at[:32, 0:1].set(W_s2).at[32:, 1:5].set(W_t2)
    bh2 = jnp.zeros((1, 8), jnp.float32)
    bh2 = bh2.at[0, 0:1].set(b_s2).at[0, 1:5].set(b_t2)

    out8 = pl.pallas_call(
        _mlp_body,
        grid=(N // BLK,),
        in_specs=[
            _row_spec(HA), _row_spec(HA),
            _full_spec((H, H)), _full_spec((1, H)),
            _full_spec((H, H)), _full_spec((1, H)),
            _full_spec((H, 64)), _full_spec((1, 64)),
            _full_spec((64, 8)), _full_spec((1, 8)),
        ],
        out_specs=_row_spec(8),
        out_shape=jax.ShapeDtypeStruct((N, 8), jnp.float32),
    )(haug, acc, W_fc1, b_fc1.reshape(1, H), W_fc2, b_fc2.reshape(1, H),
      wh1, bh1, wh2, bh2)

    return out8[:, 0], out8[:, 1:5]
